# baseline (speedup 1.0000x reference)
import jax
import jax.numpy as jnp
from jax import lax
from jax.experimental import pallas as pl
from jax.experimental.pallas import tpu as pltpu
from jax.experimental.pallas import tpu_sc as plsc

N = 10000
E = 320000
D = 128

CH = 128
NCH = E // CH
NSUB = 16
NCORE = 2
NW = NCORE * NSUB
NPAD = 10240
ROWS_PER_SUB = NPAD // NSUB

BE = 2560
BN_ = 2000


def _norm(x):
    return jnp.maximum(jnp.sqrt(jnp.sum(x * x, axis=-1, keepdims=True)), 1e-15)


def _hyp(x):
    n = _norm(x)
    e = jnp.tanh(n) * x / n
    ne = _norm(e)
    maxn = 1.0 - 1e-05
    e = jnp.where(ne > maxn, e / ne * maxn, e)
    n3 = _norm(e)
    atanh = 0.5 * (jnp.log1p(n3) - jnp.log1p(-n3))
    return atanh / n3 * e


def _elu(x):
    return jnp.where(x > 0, x, jnp.exp(jnp.minimum(x, 0.0)) - 1.0)


def _node_body(x_ref, o_ref):
    o_ref[...] = _hyp(x_ref[...])


def _node_transform(features):
    blk = pl.BlockSpec((BN_, D), lambda i: (i, 0))
    return pl.pallas_call(
        _node_body,
        grid=(N // BN_,),
        in_specs=[blk],
        out_specs=blk,
        out_shape=jax.ShapeDtypeStruct((N, D), jnp.float32),
    )(features)


NFULL = NCH // NW

SLICES = ((0, 10, 0),
          (320, 30, 0),
          (1280, 30, 0),
          (2240, 8, 4))


def _make_gather_body(c0, nfull, nrem):
    def body(feats_hbm, ei0_hbm, ei1_hbm, srcg_hbm, dstg_hbm,
             idx0a, idx1a, idx0b, idx1b, r0a, r1a, r0b, r1b,
             semi_a, semi_b, semg_a, semg_b, semw_a, semw_b):
        c = lax.axis_index("c")
        s = lax.axis_index("s")
        wid = c * NSUB + s
        idx = ((idx0a, idx1a), (idx0b, idx1b))
        rows = ((r0a, r1a), (r0b, r1b))
        semi = (semi_a, semi_b)
        semg = (semg_a, semg_b)
        semw = (semw_a, semw_b)

        def gbase(j):
            return (c0 + wid + NW * j) * CH

        def lbase(j):
            return (wid + NW * j) * CH

        for b in (0, 1):
            pb = gbase(b)
            pltpu.async_copy(ei0_hbm.at[pl.ds(pb, CH)], idx[b][0], semi[b])
            pltpu.async_copy(ei1_hbm.at[pl.ds(pb, CH)], idx[b][1], semi[b])

        def outer(g, carry):
            for b in (0, 1):
                j = g * 2 + b
                base = gbase(j)
                lb = lbase(j)

                @pl.when(j >= 2)
                def _():
                    pv = lbase(j - 2)
                    pltpu.make_async_copy(
                        rows[b][0], srcg_hbm.at[pl.ds(pv, CH)],
                        semw[b]).wait()
                    pltpu.make_async_copy(
                        rows[b][1], dstg_hbm.at[pl.ds(pv, CH)],
                        semw[b]).wait()

                pltpu.make_async_copy(
                    ei0_hbm.at[pl.ds(base, CH)], idx[b][0], semi[b]).wait()
                pltpu.make_async_copy(
                    ei1_hbm.at[pl.ds(base, CH)], idx[b][1], semi[b]).wait()
                cp0 = pltpu.async_copy(feats_hbm.at[idx[b][0]], rows[b][0],
                                       semg[b])
                cp1 = pltpu.async_copy(feats_hbm.at[idx[b][1]], rows[b][1],
                                       semg[b])
                cp0.wait()
                cp1.wait()
                pltpu.async_copy(rows[b][0], srcg_hbm.at[pl.ds(lb, CH)],
                                 semw[b])
                pltpu.async_copy(rows[b][1], dstg_hbm.at[pl.ds(lb, CH)],
                                 semw[b])

                @pl.when(j + 2 < nfull)
                def _():
                    nb = gbase(j + 2)
                    pltpu.async_copy(ei0_hbm.at[pl.ds(nb, CH)], idx[b][0],
                                     semi[b])
                    pltpu.async_copy(ei1_hbm.at[pl.ds(nb, CH)], idx[b][1],
                                     semi[b])

            return carry

        lax.fori_loop(0, nfull // 2, outer, 0)

        for b in (0, 1):
            pv = lbase(nfull - 2 + b)
            pltpu.make_async_copy(rows[b][0], srcg_hbm.at[pl.ds(pv, CH)],
                                  semw[b]).wait()
            pltpu.make_async_copy(rows[b][1], dstg_hbm.at[pl.ds(pv, CH)],
                                  semw[b]).wait()

        if nrem:
            @pl.when(wid < nrem)
            def _():
                base = (c0 + nfull * NW + wid) * CH
                lb = (nfull * NW + wid) * CH
                pltpu.sync_copy(ei0_hbm.at[pl.ds(base, CH)], idx[0][0])
                pltpu.sync_copy(ei1_hbm.at[pl.ds(base, CH)], idx[0][1])
                cp0 = pltpu.async_copy(feats_hbm.at[idx[0][0]], rows[0][0],
                                       semg[0])
                cp1 = pltpu.async_copy(feats_hbm.at[idx[0][1]], rows[0][1],
                                       semg[0])
                cp0.wait()
                cp1.wait()
                pltpu.sync_copy(rows[0][0], srcg_hbm.at[pl.ds(lb, CH)])
                pltpu.sync_copy(rows[0][1], dstg_hbm.at[pl.ds(lb, CH)])

    return body


def _make_gather(sl):
    c0, nfull, nrem = sl
    es = (nfull * NW + nrem) * CH
    return pl.kernel(
        _make_gather_body(c0, nfull, nrem),
        out_type=(jax.ShapeDtypeStruct((es, D), jnp.float32),
                  jax.ShapeDtypeStruct((es, D), jnp.float32)),
        mesh=plsc.VectorSubcoreMesh(core_axis_name="c", subcore_axis_name="s"),
        scratch_types=[
            pltpu.VMEM((CH,), jnp.int32),
            pltpu.VMEM((CH,), jnp.int32),
            pltpu.VMEM((CH,), jnp.int32),
            pltpu.VMEM((CH,), jnp.int32),
            pltpu.VMEM((CH, D), jnp.float32),
            pltpu.VMEM((CH, D), jnp.float32),
            pltpu.VMEM((CH, D), jnp.float32),
            pltpu.VMEM((CH, D), jnp.float32),
            pltpu.SemaphoreType.DMA,
            pltpu.SemaphoreType.DMA,
            pltpu.SemaphoreType.DMA,
            pltpu.SemaphoreType.DMA,
            pltpu.SemaphoreType.DMA,
            pltpu.SemaphoreType.DMA,
        ],
    )


def _edge_body(src_ref, dst_ref, w1a_ref, w1b_ref, b1_ref, lng_ref, lnb_ref,
               w2_ref, b2_ref, h2_ref, s1_ref, s2_ref):
    src = src_ref[...].astype(jnp.float32)
    dst = dst_ref[...].astype(jnp.float32)
    multi = jnp.sum(src * dst, axis=-1, keepdims=True)
    dd = src - dst
    dist = jnp.sqrt(jnp.sum(dd * dd, axis=-1, keepdims=True))
    z = 2.0 * dist + 2.0 * (dist * dist * dist / 3.0 + multi * dist * dist)
    coef = 1.0 - jax.nn.sigmoid(z)
    h = (jnp.dot((1.0 + coef) * src, w1a_ref[...],
                 preferred_element_type=jnp.float32)
         + jnp.dot(dst, w1b_ref[...], preferred_element_type=jnp.float32)
         + b1_ref[...])
    h = _elu(h)
    mu = jnp.mean(h, axis=-1, keepdims=True)
    hc = h - mu
    var = jnp.mean(hc * hc, axis=-1, keepdims=True)
    h = hc / jnp.sqrt(var + 1e-5) * lng_ref[...] + lnb_ref[...]
    h = jnp.dot(h, w2_ref[...], preferred_element_type=jnp.float32) + b2_ref[...]
    h = _elu(h)
    h2_ref[...] = h

    @pl.when(pl.program_id(0) == 0)
    def _():
        s1_ref[...] = jnp.zeros_like(s1_ref)
        s2_ref[...] = jnp.zeros_like(s2_ref)

    s1_ref[...] += jnp.sum(h, axis=0, keepdims=True)
    s2_ref[...] += jnp.sum(h * h, axis=0, keepdims=True)


def _edge_mlp(srcg, dstg, w1aT, w1bT, b1, ln_g, ln_b, w2T, b2):
    full = pl.BlockSpec((D, D), lambda i: (0, 0))
    vec = pl.BlockSpec((1, D), lambda i: (0, 0))
    return pl.pallas_call(
        _edge_body,
        grid=(srcg.shape[0] // BE,),
        in_specs=[
            pl.BlockSpec((BE, D), lambda i: (i, 0)),
            pl.BlockSpec((BE, D), lambda i: (i, 0)),
            full, full, vec, vec, vec, full, vec,
        ],
        out_specs=[
            pl.BlockSpec((BE, D), lambda i: (i, 0)),
            vec, vec,
        ],
        out_shape=[
            jax.ShapeDtypeStruct((srcg.shape[0], D), jnp.float32),
            jax.ShapeDtypeStruct((1, D), jnp.float32),
            jax.ShapeDtypeStruct((1, D), jnp.float32),
        ],
    )(srcg, dstg, w1aT, w1bT, b1, ln_g, ln_b, w2T, b2)


def _make_scatter_body(c0, nfull, nrem):
    def body(h2_hbm, ei1_hbm, zs_hbm, sp_hbm,
             idxa, idxb, h2a, h2b, s_sh,
             seml_a, seml_b, sems_a, sems_b):
        c = lax.axis_index("c")
        s = lax.axis_index("s")
        wid = c * NSUB + s
        r0 = s * ROWS_PER_SUB
        idx = (idxa, idxb)
        h2v = (h2a, h2b)
        seml = (seml_a, seml_b)
        sems = (sems_a, sems_b)

        @pl.when(s == 0)
        def _():
            pltpu.sync_copy(zs_hbm, s_sh)

        plsc.subcore_barrier()

        def gbase(j):
            return (c0 + wid + NW * j) * CH

        def lbase(j):
            return (wid + NW * j) * CH

        pltpu.async_copy(ei1_hbm.at[pl.ds(gbase(0), CH)], idx[0], seml[0])
        pltpu.async_copy(h2_hbm.at[pl.ds(lbase(0), CH)], h2v[0], seml[0])

        def outer(g, carry):
            for b in (0, 1):
                j = g * 2 + b
                pltpu.make_async_copy(
                    ei1_hbm.at[pl.ds(gbase(j), CH)], idx[b], seml[b]).wait()
                pltpu.make_async_copy(
                    h2_hbm.at[pl.ds(lbase(j), CH)], h2v[b], seml[b]).wait()
                pltpu.async_copy(h2v[b], s_sh.at[idx[b]], sems[b], add=True)

                o = 1 - b

                @pl.when(j >= 1)
                def _():
                    pltpu.make_async_copy(
                        h2v[o], s_sh.at[idx[o]], sems[o]).wait()

                @pl.when(j + 1 < nfull)
                def _():
                    pltpu.async_copy(ei1_hbm.at[pl.ds(gbase(j + 1), CH)],
                                     idx[o], seml[o])
                    pltpu.async_copy(h2_hbm.at[pl.ds(lbase(j + 1), CH)],
                                     h2v[o], seml[o])

            return carry

        lax.fori_loop(0, nfull // 2, outer, 0)
        pltpu.make_async_copy(h2v[1], s_sh.at[idx[1]], sems[1]).wait()

        if nrem:
            @pl.when(wid < nrem)
            def _():
                gb = (c0 + nfull * NW + wid) * CH
                lb = (nfull * NW + wid) * CH
                pltpu.sync_copy(ei1_hbm.at[pl.ds(gb, CH)], idx[0])
                pltpu.sync_copy(h2_hbm.at[pl.ds(lb, CH)], h2v[0])
                pltpu.sync_copy(h2v[0], s_sh.at[idx[0]], add=True)

        plsc.subcore_barrier()
        pltpu.sync_copy(s_sh.at[pl.ds(r0, ROWS_PER_SUB)],
                        sp_hbm.at[pl.ds(c * NPAD + r0, ROWS_PER_SUB)])

    return body


def _make_scatter(sl):
    c0, nfull, nrem = sl
    return pl.kernel(
        _make_scatter_body(c0, nfull, nrem),
        out_type=jax.ShapeDtypeStruct((NCORE * NPAD, D), jnp.float32),
        mesh=plsc.VectorSubcoreMesh(core_axis_name="c", subcore_axis_name="s"),
        scratch_types=[
            pltpu.VMEM((CH,), jnp.int32),
            pltpu.VMEM((CH,), jnp.int32),
            pltpu.VMEM((CH, D), jnp.float32),
            pltpu.VMEM((CH, D), jnp.float32),
            pltpu.VMEM_SHARED((NPAD, D), jnp.float32),
            pltpu.SemaphoreType.DMA,
            pltpu.SemaphoreType.DMA,
            pltpu.SemaphoreType.DMA,
            pltpu.SemaphoreType.DMA,
        ],
    )


def _make_scatter2_body(slA, slB):
    c0a, nfa, nra = slA
    c0b, nfb, nrb = slB

    def body(h2a_hbm, h2b_hbm, ei1_hbm, zs_hbm, sp_hbm,
             idxa, idxb, h2a, h2b, s_sh,
             seml_a, seml_b, sems_a, sems_b):
        c = lax.axis_index("c")
        s = lax.axis_index("s")
        wid = c * NSUB + s
        r0 = s * ROWS_PER_SUB
        idx = (idxa, idxb)
        h2v = (h2a, h2b)
        seml = (seml_a, seml_b)
        sems = (sems_a, sems_b)

        @pl.when(s == 0)
        def _():
            pltpu.sync_copy(zs_hbm, s_sh)

        plsc.subcore_barrier()

        def one_slice(h2_hbm, c0, nfull, nrem):
            def gbase(j):
                return (c0 + wid + NW * j) * CH

            def lbase(j):
                return (wid + NW * j) * CH

            pltpu.async_copy(ei1_hbm.at[pl.ds(gbase(0), CH)], idx[0],
                             seml[0])
            pltpu.async_copy(h2_hbm.at[pl.ds(lbase(0), CH)], h2v[0],
                             seml[0])

            def outer(g, carry):
                for b in (0, 1):
                    j = g * 2 + b
                    pltpu.make_async_copy(
                        ei1_hbm.at[pl.ds(gbase(j), CH)], idx[b],
                        seml[b]).wait()
                    pltpu.make_async_copy(
                        h2_hbm.at[pl.ds(lbase(j), CH)], h2v[b],
                        seml[b]).wait()
                    pltpu.async_copy(h2v[b], s_sh.at[idx[b]], sems[b],
                                     add=True)

                    o = 1 - b

                    @pl.when(j >= 1)
                    def _():
                        pltpu.make_async_copy(
                            h2v[o], s_sh.at[idx[o]], sems[o]).wait()

                    @pl.when(j + 1 < nfull)
                    def _():
                        pltpu.async_copy(
                            ei1_hbm.at[pl.ds(gbase(j + 1), CH)], idx[o],
                            seml[o])
                        pltpu.async_copy(
                            h2_hbm.at[pl.ds(lbase(j + 1), CH)], h2v[o],
                            seml[o])

                return carry

            lax.fori_loop(0, nfull // 2, outer, 0)
            pltpu.make_async_copy(h2v[1], s_sh.at[idx[1]], sems[1]).wait()

            if nrem:
                @pl.when(wid < nrem)
                def _():
                    gb = (c0 + nfull * NW + wid) * CH
                    lb = (nfull * NW + wid) * CH
                    pltpu.sync_copy(ei1_hbm.at[pl.ds(gb, CH)], idx[0])
                    pltpu.sync_copy(h2_hbm.at[pl.ds(lb, CH)], h2v[0])
                    pltpu.sync_copy(h2v[0], s_sh.at[idx[0]], add=True)

        one_slice(h2a_hbm, c0a, nfa, nra)
        one_slice(h2b_hbm, c0b, nfb, nrb)

        plsc.subcore_barrier()
        pltpu.sync_copy(s_sh.at[pl.ds(r0, ROWS_PER_SUB)],
                        sp_hbm.at[pl.ds(c * NPAD + r0, ROWS_PER_SUB)])

    return body


def _make_scatter2(slA, slB):
    return pl.kernel(
        _make_scatter2_body(slA, slB),
        out_type=jax.ShapeDtypeStruct((NCORE * NPAD, D), jnp.float32),
        mesh=plsc.VectorSubcoreMesh(core_axis_name="c", subcore_axis_name="s"),
        scratch_types=[
            pltpu.VMEM((CH,), jnp.int32),
            pltpu.VMEM((CH,), jnp.int32),
            pltpu.VMEM((CH, D), jnp.float32),
            pltpu.VMEM((CH, D), jnp.float32),
            pltpu.VMEM_SHARED((NPAD, D), jnp.float32),
            pltpu.SemaphoreType.DMA,
            pltpu.SemaphoreType.DMA,
            pltpu.SemaphoreType.DMA,
            pltpu.SemaphoreType.DMA,
        ],
    )


def _deg_body(ei1_hbm, zs_hbm, ones_hbm, dg_hbm,
              idxa, idxb, ones_v, s_sh, seml_a, seml_b, sems_a, sems_b):
    c = lax.axis_index("c")
    s = lax.axis_index("s")
    wid = c * NSUB + s
    r0 = s * ROWS_PER_SUB
    idx = (idxa, idxb)
    seml = (seml_a, seml_b)
    sems = (sems_a, sems_b)

    @pl.when(s == 0)
    def _():
        pltpu.sync_copy(zs_hbm, s_sh)

    pltpu.sync_copy(ones_hbm, ones_v)
    plsc.subcore_barrier()

    def gbase(j):
        return (wid + NW * j) * CH

    pltpu.async_copy(ei1_hbm.at[pl.ds(gbase(0), CH)], idx[0], seml[0])

    def douter(g, carry):
        for b in (0, 1):
            j = g * 2 + b
            pltpu.make_async_copy(
                ei1_hbm.at[pl.ds(gbase(j), CH)], idx[b], seml[b]).wait()
            pltpu.async_copy(ones_v, s_sh.at[idx[b]], sems[b], add=True)

            o = 1 - b

            @pl.when(j >= 1)
            def _():
                pltpu.make_async_copy(
                    ones_v, s_sh.at[idx[o]], sems[o]).wait()

            @pl.when(j + 1 < NFULL)
            def _():
                pltpu.async_copy(ei1_hbm.at[pl.ds(gbase(j + 1), CH)],
                                 idx[o], seml[o])

        return carry

    lax.fori_loop(0, NFULL // 2, douter, 0)
    pltpu.make_async_copy(ones_v, s_sh.at[idx[1]], sems[1]).wait()

    @pl.when(wid < NCH - NFULL * NW)
    def _():
        gb = (NFULL * NW + wid) * CH
        pltpu.sync_copy(ei1_hbm.at[pl.ds(gb, CH)], idx[0])
        pltpu.sync_copy(ones_v, s_sh.at[idx[0]], add=True)

    plsc.subcore_barrier()
    pltpu.sync_copy(s_sh.at[pl.ds(r0, ROWS_PER_SUB)],
                    dg_hbm.at[pl.ds(c * NPAD + r0, ROWS_PER_SUB)])


def _deg(ei1, zs, ones):
    f = pl.kernel(
        _deg_body,
        out_type=jax.ShapeDtypeStruct((NCORE * NPAD, D), jnp.float32),
        mesh=plsc.VectorSubcoreMesh(core_axis_name="c", subcore_axis_name="s"),
        scratch_types=[
            pltpu.VMEM((CH,), jnp.int32),
            pltpu.VMEM((CH,), jnp.int32),
            pltpu.VMEM((CH, D), jnp.float32),
            pltpu.VMEM_SHARED((NPAD, D), jnp.float32),
            pltpu.SemaphoreType.DMA,
            pltpu.SemaphoreType.DMA,
            pltpu.SemaphoreType.DMA,
            pltpu.SemaphoreType.DMA,
        ],
    )
    return f(ei1, zs, ones)


def _final_body(sp0_ref, sp1_ref, dg_ref, s1_ref, s2_ref,
                bng_ref, bnb_ref, wo_ref, bo_ref, feats_ref, o_ref):
    s_sum = sp0_ref[0] + sp0_ref[1] + sp1_ref[0] + sp1_ref[1]
    d_sum = dg_ref[0] + dg_ref[1]
    m = s1_ref[...] / float(E)
    v = s2_ref[...] / float(E) - m * m
    a = bng_ref[...] / jnp.sqrt(v + 1e-5)
    cv = bnb_ref[...] - m * a
    kv = jnp.dot(cv, wo_ref[...], preferred_element_type=jnp.float32)
    out = (jnp.dot(s_sum * a, wo_ref[...], preferred_element_type=jnp.float32)
           + d_sum * kv + bo_ref[...])
    out = _hyp(out)
    out = 1.0507009873554805 * jnp.where(
        out > 0, out, 1.6732632423543772 * (jnp.exp(jnp.minimum(out, 0.0)) - 1.0))
    o_ref[...] = out + feats_ref[...]


def _final(sps, dg, s1, s2, bn_g, bn_b, woT, bo, feats):
    vec = pl.BlockSpec((1, D), lambda i: (0, 0))
    acc = pl.BlockSpec((NCORE, BN_, D), lambda i: (0, i, 0))
    return pl.pallas_call(
        _final_body,
        grid=(N // BN_,),
        in_specs=[
            acc, acc, acc,
            vec, vec, vec, vec,
            pl.BlockSpec((D, D), lambda i: (0, 0)),
            vec,
            pl.BlockSpec((BN_, D), lambda i: (i, 0)),
        ],
        out_specs=pl.BlockSpec((BN_, D), lambda i: (i, 0)),
        out_shape=jax.ShapeDtypeStruct((N, D), jnp.float32),
    )(sps[0], sps[1], dg, s1, s2, bn_g, bn_b, woT, bo, feats)


def kernel(features, edge_index, c, W1, b1, ln_g, ln_b, W2, b2, bn_g, bn_b,
           Wo, bo):
    del c
    f32 = jnp.float32

    feats = _node_transform(features)

    ei0 = edge_index[0]
    ei1 = edge_index[1]
    gathered = [_make_gather(sl)(feats, ei0, ei1) for sl in SLICES]

    w1aT = W1[:, :D].T
    w1bT = W1[:, D:].T
    w2T = W2.T
    mlp = [_edge_mlp(sg, dg_, w1aT, w1bT, b1[None], ln_g[None], ln_b[None],
                     w2T, b2[None]) for sg, dg_ in gathered]

    zs = jnp.zeros((NPAD, D), f32)
    ones = jnp.ones((CH, D), f32)
    sp0 = _make_scatter2(SLICES[0], SLICES[1])(mlp[0][0], mlp[1][0], ei1, zs)
    dg = _deg(ei1, zs, ones)
    sp12 = _make_scatter2(SLICES[2], SLICES[3])(mlp[2][0], mlp[3][0], ei1, zs)

    sp0 = sp0.reshape(NCORE, NPAD, D)
    sp12 = sp12.reshape(NCORE, NPAD, D)
    dg = dg.reshape(NCORE, NPAD, D)
    s1 = mlp[0][1] + mlp[1][1] + mlp[2][1] + mlp[3][1]
    s2 = mlp[0][2] + mlp[1][2] + mlp[2][2] + mlp[3][2]
    return _final([sp0, sp12], dg, s1, s2, bn_g[None], bn_b[None], Wo.T,
                  bo[None], feats)


# speedup vs baseline: 1.0319x; 1.0319x over previous
import jax
import jax.numpy as jnp
from jax import lax
from jax.experimental import pallas as pl
from jax.experimental.pallas import tpu as pltpu
from jax.experimental.pallas import tpu_sc as plsc

N = 10000
E = 320000
D = 128

CH = 128
NCH = E // CH
NSUB = 16
NCORE = 2
NW = NCORE * NSUB
NPAD = 10240
ROWS_PER_SUB = NPAD // NSUB

BE = 2560
BN_ = 2000


def _norm(x):
    return jnp.maximum(jnp.sqrt(jnp.sum(x * x, axis=-1, keepdims=True)), 1e-15)


def _hyp(x):
    n = _norm(x)
    e = jnp.tanh(n) * x / n
    ne = _norm(e)
    maxn = 1.0 - 1e-05
    e = jnp.where(ne > maxn, e / ne * maxn, e)
    n3 = _norm(e)
    atanh = 0.5 * (jnp.log1p(n3) - jnp.log1p(-n3))
    return atanh / n3 * e


def _elu(x):
    return jnp.where(x > 0, x, jnp.exp(jnp.minimum(x, 0.0)) - 1.0)


def _node_body(x_ref, o_ref):
    o_ref[...] = _hyp(x_ref[...])


def _node_transform(features):
    blk = pl.BlockSpec((BN_, D), lambda i: (i, 0))
    return pl.pallas_call(
        _node_body,
        grid=(N // BN_,),
        in_specs=[blk],
        out_specs=blk,
        out_shape=jax.ShapeDtypeStruct((N, D), jnp.float32),
    )(features)


NFULL = NCH // NW

SLICES = ((0, 20, 0),
          (640, 30, 0),
          (1600, 28, 4))


def _make_gather_body(c0, nfull, nrem):
    def body(feats_hbm, ei0_hbm, ei1_hbm, srcg_hbm, dstg_hbm,
             idx0a, idx1a, idx0b, idx1b, r0a, r1a, r0b, r1b,
             semi_a, semi_b, semg_a, semg_b, semw_a, semw_b):
        c = lax.axis_index("c")
        s = lax.axis_index("s")
        wid = c * NSUB + s
        idx = ((idx0a, idx1a), (idx0b, idx1b))
        rows = ((r0a, r1a), (r0b, r1b))
        semi = (semi_a, semi_b)
        semg = (semg_a, semg_b)
        semw = (semw_a, semw_b)

        def gbase(j):
            return (c0 + wid + NW * j) * CH

        def lbase(j):
            return (wid + NW * j) * CH

        for b in (0, 1):
            pb = gbase(b)
            pltpu.async_copy(ei0_hbm.at[pl.ds(pb, CH)], idx[b][0], semi[b])
            pltpu.async_copy(ei1_hbm.at[pl.ds(pb, CH)], idx[b][1], semi[b])

        def outer(g, carry):
            for b in (0, 1):
                j = g * 2 + b
                base = gbase(j)
                lb = lbase(j)

                @pl.when(j >= 2)
                def _():
                    pv = lbase(j - 2)
                    pltpu.make_async_copy(
                        rows[b][0], srcg_hbm.at[pl.ds(pv, CH)],
                        semw[b]).wait()
                    pltpu.make_async_copy(
                        rows[b][1], dstg_hbm.at[pl.ds(pv, CH)],
                        semw[b]).wait()

                pltpu.make_async_copy(
                    ei0_hbm.at[pl.ds(base, CH)], idx[b][0], semi[b]).wait()
                pltpu.make_async_copy(
                    ei1_hbm.at[pl.ds(base, CH)], idx[b][1], semi[b]).wait()
                cp0 = pltpu.async_copy(feats_hbm.at[idx[b][0]], rows[b][0],
                                       semg[b])
                cp1 = pltpu.async_copy(feats_hbm.at[idx[b][1]], rows[b][1],
                                       semg[b])
                cp0.wait()
                cp1.wait()
                pltpu.async_copy(rows[b][0], srcg_hbm.at[pl.ds(lb, CH)],
                                 semw[b])
                pltpu.async_copy(rows[b][1], dstg_hbm.at[pl.ds(lb, CH)],
                                 semw[b])

                @pl.when(j + 2 < nfull)
                def _():
                    nb = gbase(j + 2)
                    pltpu.async_copy(ei0_hbm.at[pl.ds(nb, CH)], idx[b][0],
                                     semi[b])
                    pltpu.async_copy(ei1_hbm.at[pl.ds(nb, CH)], idx[b][1],
                                     semi[b])

            return carry

        lax.fori_loop(0, nfull // 2, outer, 0)

        for b in (0, 1):
            pv = lbase(nfull - 2 + b)
            pltpu.make_async_copy(rows[b][0], srcg_hbm.at[pl.ds(pv, CH)],
                                  semw[b]).wait()
            pltpu.make_async_copy(rows[b][1], dstg_hbm.at[pl.ds(pv, CH)],
                                  semw[b]).wait()

        if nrem:
            @pl.when(wid < nrem)
            def _():
                base = (c0 + nfull * NW + wid) * CH
                lb = (nfull * NW + wid) * CH
                pltpu.sync_copy(ei0_hbm.at[pl.ds(base, CH)], idx[0][0])
                pltpu.sync_copy(ei1_hbm.at[pl.ds(base, CH)], idx[0][1])
                cp0 = pltpu.async_copy(feats_hbm.at[idx[0][0]], rows[0][0],
                                       semg[0])
                cp1 = pltpu.async_copy(feats_hbm.at[idx[0][1]], rows[0][1],
                                       semg[0])
                cp0.wait()
                cp1.wait()
                pltpu.sync_copy(rows[0][0], srcg_hbm.at[pl.ds(lb, CH)])
                pltpu.sync_copy(rows[0][1], dstg_hbm.at[pl.ds(lb, CH)])

    return body


def _make_gather(sl):
    c0, nfull, nrem = sl
    es = (nfull * NW + nrem) * CH
    return pl.kernel(
        _make_gather_body(c0, nfull, nrem),
        out_type=(jax.ShapeDtypeStruct((es, D), jnp.float32),
                  jax.ShapeDtypeStruct((es, D), jnp.float32)),
        mesh=plsc.VectorSubcoreMesh(core_axis_name="c", subcore_axis_name="s"),
        scratch_types=[
            pltpu.VMEM((CH,), jnp.int32),
            pltpu.VMEM((CH,), jnp.int32),
            pltpu.VMEM((CH,), jnp.int32),
            pltpu.VMEM((CH,), jnp.int32),
            pltpu.VMEM((CH, D), jnp.float32),
            pltpu.VMEM((CH, D), jnp.float32),
            pltpu.VMEM((CH, D), jnp.float32),
            pltpu.VMEM((CH, D), jnp.float32),
            pltpu.SemaphoreType.DMA,
            pltpu.SemaphoreType.DMA,
            pltpu.SemaphoreType.DMA,
            pltpu.SemaphoreType.DMA,
            pltpu.SemaphoreType.DMA,
            pltpu.SemaphoreType.DMA,
        ],
    )


def _edge_body(src_ref, dst_ref, w1a_ref, w1b_ref, b1_ref, lng_ref, lnb_ref,
               w2_ref, b2_ref, h2_ref, s1_ref, s2_ref):
    src = src_ref[...].astype(jnp.float32)
    dst = dst_ref[...].astype(jnp.float32)
    multi = jnp.sum(src * dst, axis=-1, keepdims=True)
    dd = src - dst
    dist = jnp.sqrt(jnp.sum(dd * dd, axis=-1, keepdims=True))
    z = 2.0 * dist + 2.0 * (dist * dist * dist / 3.0 + multi * dist * dist)
    coef = 1.0 - jax.nn.sigmoid(z)
    h = (jnp.dot((1.0 + coef) * src, w1a_ref[...],
                 preferred_element_type=jnp.float32)
         + jnp.dot(dst, w1b_ref[...], preferred_element_type=jnp.float32)
         + b1_ref[...])
    h = _elu(h)
    mu = jnp.mean(h, axis=-1, keepdims=True)
    hc = h - mu
    var = jnp.mean(hc * hc, axis=-1, keepdims=True)
    h = hc / jnp.sqrt(var + 1e-5) * lng_ref[...] + lnb_ref[...]
    h = jnp.dot(h, w2_ref[...], preferred_element_type=jnp.float32) + b2_ref[...]
    h = _elu(h)
    h2_ref[...] = h

    @pl.when(pl.program_id(0) == 0)
    def _():
        s1_ref[...] = jnp.zeros_like(s1_ref)
        s2_ref[...] = jnp.zeros_like(s2_ref)

    s1_ref[...] += jnp.sum(h, axis=0, keepdims=True)
    s2_ref[...] += jnp.sum(h * h, axis=0, keepdims=True)


def _edge_mlp(srcg, dstg, w1aT, w1bT, b1, ln_g, ln_b, w2T, b2):
    full = pl.BlockSpec((D, D), lambda i: (0, 0))
    vec = pl.BlockSpec((1, D), lambda i: (0, 0))
    return pl.pallas_call(
        _edge_body,
        grid=(srcg.shape[0] // BE,),
        in_specs=[
            pl.BlockSpec((BE, D), lambda i: (i, 0)),
            pl.BlockSpec((BE, D), lambda i: (i, 0)),
            full, full, vec, vec, vec, full, vec,
        ],
        out_specs=[
            pl.BlockSpec((BE, D), lambda i: (i, 0)),
            vec, vec,
        ],
        out_shape=[
            jax.ShapeDtypeStruct((srcg.shape[0], D), jnp.float32),
            jax.ShapeDtypeStruct((1, D), jnp.float32),
            jax.ShapeDtypeStruct((1, D), jnp.float32),
        ],
    )(srcg, dstg, w1aT, w1bT, b1, ln_g, ln_b, w2T, b2)


def _make_scatter_body(c0, nfull, nrem):
    def body(h2_hbm, ei1_hbm, zs_hbm, sp_hbm,
             idxa, idxb, h2a, h2b, s_sh,
             seml_a, seml_b, sems_a, sems_b):
        c = lax.axis_index("c")
        s = lax.axis_index("s")
        wid = c * NSUB + s
        r0 = s * ROWS_PER_SUB
        idx = (idxa, idxb)
        h2v = (h2a, h2b)
        seml = (seml_a, seml_b)
        sems = (sems_a, sems_b)

        @pl.when(s == 0)
        def _():
            pltpu.sync_copy(zs_hbm, s_sh)

        plsc.subcore_barrier()

        def gbase(j):
            return (c0 + wid + NW * j) * CH

        def lbase(j):
            return (wid + NW * j) * CH

        pltpu.async_copy(ei1_hbm.at[pl.ds(gbase(0), CH)], idx[0], seml[0])
        pltpu.async_copy(h2_hbm.at[pl.ds(lbase(0), CH)], h2v[0], seml[0])

        def outer(g, carry):
            for b in (0, 1):
                j = g * 2 + b
                pltpu.make_async_copy(
                    ei1_hbm.at[pl.ds(gbase(j), CH)], idx[b], seml[b]).wait()
                pltpu.make_async_copy(
                    h2_hbm.at[pl.ds(lbase(j), CH)], h2v[b], seml[b]).wait()
                pltpu.async_copy(h2v[b], s_sh.at[idx[b]], sems[b], add=True)

                o = 1 - b

                @pl.when(j >= 1)
                def _():
                    pltpu.make_async_copy(
                        h2v[o], s_sh.at[idx[o]], sems[o]).wait()

                @pl.when(j + 1 < nfull)
                def _():
                    pltpu.async_copy(ei1_hbm.at[pl.ds(gbase(j + 1), CH)],
                                     idx[o], seml[o])
                    pltpu.async_copy(h2_hbm.at[pl.ds(lbase(j + 1), CH)],
                                     h2v[o], seml[o])

            return carry

        lax.fori_loop(0, nfull // 2, outer, 0)
        pltpu.make_async_copy(h2v[1], s_sh.at[idx[1]], sems[1]).wait()

        if nrem:
            @pl.when(wid < nrem)
            def _():
                gb = (c0 + nfull * NW + wid) * CH
                lb = (nfull * NW + wid) * CH
                pltpu.sync_copy(ei1_hbm.at[pl.ds(gb, CH)], idx[0])
                pltpu.sync_copy(h2_hbm.at[pl.ds(lb, CH)], h2v[0])
                pltpu.sync_copy(h2v[0], s_sh.at[idx[0]], add=True)

        plsc.subcore_barrier()
        pltpu.sync_copy(s_sh.at[pl.ds(r0, ROWS_PER_SUB)],
                        sp_hbm.at[pl.ds(c * NPAD + r0, ROWS_PER_SUB)])

    return body


def _make_scatter(sl):
    c0, nfull, nrem = sl
    return pl.kernel(
        _make_scatter_body(c0, nfull, nrem),
        out_type=jax.ShapeDtypeStruct((NCORE * NPAD, D), jnp.float32),
        mesh=plsc.VectorSubcoreMesh(core_axis_name="c", subcore_axis_name="s"),
        scratch_types=[
            pltpu.VMEM((CH,), jnp.int32),
            pltpu.VMEM((CH,), jnp.int32),
            pltpu.VMEM((CH, D), jnp.float32),
            pltpu.VMEM((CH, D), jnp.float32),
            pltpu.VMEM_SHARED((NPAD, D), jnp.float32),
            pltpu.SemaphoreType.DMA,
            pltpu.SemaphoreType.DMA,
            pltpu.SemaphoreType.DMA,
            pltpu.SemaphoreType.DMA,
        ],
    )


def _deg_body(ei1_hbm, zs_hbm, ones_hbm, dg_hbm,
              idxa, idxb, ones_v, s_sh, seml_a, seml_b, sems_a, sems_b):
    c = lax.axis_index("c")
    s = lax.axis_index("s")
    wid = c * NSUB + s
    r0 = s * ROWS_PER_SUB
    idx = (idxa, idxb)
    seml = (seml_a, seml_b)
    sems = (sems_a, sems_b)

    @pl.when(s == 0)
    def _():
        pltpu.sync_copy(zs_hbm, s_sh)

    pltpu.sync_copy(ones_hbm, ones_v)
    plsc.subcore_barrier()

    def gbase(j):
        return (wid + NW * j) * CH

    pltpu.async_copy(ei1_hbm.at[pl.ds(gbase(0), CH)], idx[0], seml[0])

    def douter(g, carry):
        for b in (0, 1):
            j = g * 2 + b
            pltpu.make_async_copy(
                ei1_hbm.at[pl.ds(gbase(j), CH)], idx[b], seml[b]).wait()
            pltpu.async_copy(ones_v, s_sh.at[idx[b]], sems[b], add=True)

            o = 1 - b

            @pl.when(j >= 1)
            def _():
                pltpu.make_async_copy(
                    ones_v, s_sh.at[idx[o]], sems[o]).wait()

            @pl.when(j + 1 < NFULL)
            def _():
                pltpu.async_copy(ei1_hbm.at[pl.ds(gbase(j + 1), CH)],
                                 idx[o], seml[o])

        return carry

    lax.fori_loop(0, NFULL // 2, douter, 0)
    pltpu.make_async_copy(ones_v, s_sh.at[idx[1]], sems[1]).wait()

    @pl.when(wid < NCH - NFULL * NW)
    def _():
        gb = (NFULL * NW + wid) * CH
        pltpu.sync_copy(ei1_hbm.at[pl.ds(gb, CH)], idx[0])
        pltpu.sync_copy(ones_v, s_sh.at[idx[0]], add=True)

    plsc.subcore_barrier()
    pltpu.sync_copy(s_sh.at[pl.ds(r0, ROWS_PER_SUB)],
                    dg_hbm.at[pl.ds(c * NPAD + r0, ROWS_PER_SUB)])


def _deg(ei1, zs, ones):
    f = pl.kernel(
        _deg_body,
        out_type=jax.ShapeDtypeStruct((NCORE * NPAD, D), jnp.float32),
        mesh=plsc.VectorSubcoreMesh(core_axis_name="c", subcore_axis_name="s"),
        scratch_types=[
            pltpu.VMEM((CH,), jnp.int32),
            pltpu.VMEM((CH,), jnp.int32),
            pltpu.VMEM((CH, D), jnp.float32),
            pltpu.VMEM_SHARED((NPAD, D), jnp.float32),
            pltpu.SemaphoreType.DMA,
            pltpu.SemaphoreType.DMA,
            pltpu.SemaphoreType.DMA,
            pltpu.SemaphoreType.DMA,
        ],
    )
    return f(ei1, zs, ones)


def _final_body(sp0_ref, sp1_ref, sp2_ref, dg_ref, s1_ref, s2_ref,
                bng_ref, bnb_ref, wo_ref, bo_ref, feats_ref, o_ref):
    s_sum = (sp0_ref[0] + sp0_ref[1] + sp1_ref[0] + sp1_ref[1]
             + sp2_ref[0] + sp2_ref[1])
    d_sum = dg_ref[0] + dg_ref[1]
    m = s1_ref[...] / float(E)
    v = s2_ref[...] / float(E) - m * m
    a = bng_ref[...] / jnp.sqrt(v + 1e-5)
    cv = bnb_ref[...] - m * a
    kv = jnp.dot(cv, wo_ref[...], preferred_element_type=jnp.float32)
    out = (jnp.dot(s_sum * a, wo_ref[...], preferred_element_type=jnp.float32)
           + d_sum * kv + bo_ref[...])
    out = _hyp(out)
    out = 1.0507009873554805 * jnp.where(
        out > 0, out, 1.6732632423543772 * (jnp.exp(jnp.minimum(out, 0.0)) - 1.0))
    o_ref[...] = out + feats_ref[...]


def _final(sps, dg, s1, s2, bn_g, bn_b, woT, bo, feats):
    vec = pl.BlockSpec((1, D), lambda i: (0, 0))
    acc = pl.BlockSpec((NCORE, BN_, D), lambda i: (0, i, 0))
    return pl.pallas_call(
        _final_body,
        grid=(N // BN_,),
        in_specs=[
            acc, acc, acc, acc,
            vec, vec, vec, vec,
            pl.BlockSpec((D, D), lambda i: (0, 0)),
            vec,
            pl.BlockSpec((BN_, D), lambda i: (i, 0)),
        ],
        out_specs=pl.BlockSpec((BN_, D), lambda i: (i, 0)),
        out_shape=jax.ShapeDtypeStruct((N, D), jnp.float32),
    )(sps[0], sps[1], sps[2], dg, s1, s2, bn_g, bn_b, woT, bo, feats)


def kernel(features, edge_index, c, W1, b1, ln_g, ln_b, W2, b2, bn_g, bn_b,
           Wo, bo):
    del c
    f32 = jnp.float32

    feats = _node_transform(features)

    ei0 = edge_index[0]
    ei1 = edge_index[1]
    gathered = [_make_gather(sl)(feats, ei0, ei1) for sl in SLICES]

    w1aT = W1[:, :D].T
    w1bT = W1[:, D:].T
    w2T = W2.T
    mlp = [_edge_mlp(sg, dg_, w1aT, w1bT, b1[None], ln_g[None], ln_b[None],
                     w2T, b2[None]) for sg, dg_ in gathered]

    zs = jnp.zeros((NPAD, D), f32)
    ones = jnp.ones((CH, D), f32)
    sp0 = _make_scatter(SLICES[0])(mlp[0][0], ei1, zs)
    dg = _deg(ei1, zs, ones)
    sp1 = _make_scatter(SLICES[1])(mlp[1][0], ei1, zs)
    sp2 = _make_scatter(SLICES[2])(mlp[2][0], ei1, zs)

    sp0 = sp0.reshape(NCORE, NPAD, D)
    sp1 = sp1.reshape(NCORE, NPAD, D)
    sp2 = sp2.reshape(NCORE, NPAD, D)
    dg = dg.reshape(NCORE, NPAD, D)
    s1 = mlp[0][1] + mlp[1][1] + mlp[2][1]
    s2 = mlp[0][2] + mlp[1][2] + mlp[2][2]
    return _final([sp0, sp1, sp2], dg, s1, s2, bn_g[None], bn_b[None], Wo.T,
                  bo[None], feats)
